# baseline (speedup 1.0000x reference)
import functools

import jax
import jax.numpy as jnp
from jax import lax
from jax.experimental import pallas as pl
from jax.experimental.pallas import tpu as pltpu
from jax.experimental.pallas import tpu_sc as plsc

NC = 2
NS = 16
NPAD = 102400
RB = 6400
W_EDGE = 2000
D = 8


def _sc_mesh():
    return plsc.VectorSubcoreMesh(
        core_axis_name="c", subcore_axis_name="s", num_cores=NC,
        num_subcores=NS)


_SC_PARAMS = pltpu.CompilerParams(use_tc_tiling_on_sc=False)


def _sc_degree(dst, zeros_n):
    E = dst.shape[0]
    ept = E // (NC * NS)
    nwin = ept // W_EDGE
    assert nwin * W_EDGE == ept
    rpt = NPAD // NS

    assert nwin % 4 == 0 and nwin >= 8

    @functools.partial(
        pl.kernel,
        out_type=jax.ShapeDtypeStruct((NC, NPAD), jnp.float32),
        mesh=_sc_mesh(),
        scratch_types=[
            pltpu.VMEM_SHARED((NPAD,), jnp.float32),
            [pltpu.VMEM((W_EDGE,), jnp.int32) for _ in range(4)],
            pltpu.VMEM((W_EDGE,), jnp.float32),
            [pltpu.SemaphoreType.DMA for _ in range(4)],
            [pltpu.SemaphoreType.DMA for _ in range(4)],
        ],
        compiler_params=_SC_PARAMS,
    )
    def deg_kernel(dst_hbm, zeros_hbm, out_hbm, deg_s, dbufs, ones,
                   isems, ssems):
        c = lax.axis_index("c")
        s = lax.axis_index("s")
        sl = pl.ds(s * rpt, rpt)
        pltpu.sync_copy(zeros_hbm.at[sl], deg_s.at[sl])

        def fill(i, _):
            ones[pl.ds(i * 16, 16)] = jnp.full((16,), 1.0, jnp.float32)
            return 0
        lax.fori_loop(0, W_EDGE // 16, fill, 0)
        plsc.subcore_barrier()

        base0 = (c * NS + s) * ept

        def istart(w, p):
            b = base0 + w * W_EDGE
            pltpu.async_copy(dst_hbm.at[pl.ds(b, W_EDGE)], dbufs[p],
                             isems[p])

        def sdesc(p):
            return pltpu.make_async_copy(ones, deg_s.at[dbufs[p]], ssems[p])

        istart(0, 0)
        istart(1, 1)

        def grp(g, _):
            for ph in range(4):
                w = g * 4 + ph
                p = ph
                q = (ph + 2) % 4
                pltpu.make_async_copy(
                    dst_hbm.at[pl.ds(base0, W_EDGE)], dbufs[p],
                    isems[p]).wait()
                pltpu.async_copy(ones, deg_s.at[dbufs[p]], ssems[p],
                                 add=True)

                @pl.when(w >= 2)
                def _():
                    sdesc(q).wait()

                @pl.when(w + 2 < nwin)
                def _():
                    istart(w + 2, q)
            return 0
        lax.fori_loop(0, nwin // 4, grp, 0)
        sdesc((nwin - 2) % 4).wait()
        sdesc((nwin - 1) % 4).wait()
        plsc.subcore_barrier()
        pltpu.sync_copy(deg_s.at[sl], out_hbm.at[c, sl])

    return deg_kernel(dst, zeros_n)


def _sc_edge_pass(src, dst, y, zeros_nd, dpack):
    E = src.shape[0]
    ept = E // (NC * NS)
    nwin = ept // W_EDGE
    assert nwin * W_EDGE == ept
    rpt = NPAD // NS
    assert nwin % 2 == 0 and nwin >= 4

    @functools.partial(
        pl.kernel,
        out_type=jax.ShapeDtypeStruct((NC, NPAD, D), jnp.float32),
        mesh=_sc_mesh(),
        scratch_types=[
            pltpu.VMEM_SHARED((NPAD, D), jnp.float32),
            pltpu.VMEM_SHARED((NPAD, D), jnp.float32),
            [pltpu.VMEM((W_EDGE,), jnp.int32) for _ in range(2)],
            [pltpu.VMEM((W_EDGE,), jnp.int32) for _ in range(2)],
            pltpu.VMEM((W_EDGE, D), jnp.float32),
            [pltpu.SemaphoreType.DMA for _ in range(2)],
            pltpu.SemaphoreType.DMA,
        ],
        compiler_params=_SC_PARAMS,
    )
    def edge_kernel(src_hbm, dst_hbm, y_hbm, zeros_hbm, out_hbm,
                    y_s, agg_s, sbufs, dbufs, rows, isems, gsem):
        c = lax.axis_index("c")
        s = lax.axis_index("s")
        sl = pl.ds(s * rpt, rpt)
        pltpu.sync_copy(y_hbm.at[sl], y_s.at[sl])
        pltpu.sync_copy(zeros_hbm.at[sl], agg_s.at[sl])
        plsc.subcore_barrier()

        base0 = (c * NS + s) * ept

        def istart(w, p):
            b = base0 + w * W_EDGE
            pltpu.async_copy(src_hbm.at[pl.ds(b, W_EDGE)], sbufs[p],
                             isems[p])
            pltpu.async_copy(dst_hbm.at[pl.ds(b, W_EDGE)], dbufs[p],
                             isems[p])

        def iwait(p):
            pltpu.make_async_copy(
                src_hbm.at[pl.ds(base0, W_EDGE)], sbufs[p], isems[p]).wait()
            pltpu.make_async_copy(
                dst_hbm.at[pl.ds(base0, W_EDGE)], dbufs[p], isems[p]).wait()

        istart(0, 0)

        def grp(g, _):
            for ph in range(2):
                w = g * 2 + ph
                iwait(ph)

                @pl.when(w + 1 < nwin)
                def _():
                    istart(w + 1, 1 - ph)

                pltpu.async_copy(y_s.at[sbufs[ph]], rows, gsem).wait()
                pltpu.sync_copy(rows, agg_s.at[dbufs[ph]], add=True)
            return 0
        lax.fori_loop(0, nwin // 2, grp, 0)
        plsc.subcore_barrier()
        pltpu.sync_copy(agg_s.at[sl], out_hbm.at[c, sl])

    return edge_kernel(src, dst, y, zeros_nd)


def _tc_head(degp, xp, W1p):
    grid = NPAD // RB

    def body(degp_ref, x_ref, w_ref, dinv_ref, y_ref):
        ones2 = jnp.ones((2, 1), jnp.float32)
        deg = lax.dot_general(degp_ref[...], ones2,
                              (((0,), (0,)), ((), ()))) + 1.0
        dinv = lax.rsqrt(deg)
        dinv_ref[...] = dinv
        y_ref[...] = jnp.dot(x_ref[...], w_ref[...]) * dinv

    return pl.pallas_call(
        body,
        grid=(grid,),
        in_specs=[
            pl.BlockSpec((2, RB), lambda i: (0, i)),
            pl.BlockSpec((RB, D), lambda i: (i, 0)),
            pl.BlockSpec((D, D), lambda i: (0, 0)),
        ],
        out_specs=[
            pl.BlockSpec((RB, 1), lambda i: (i, 0)),
            pl.BlockSpec((RB, D), lambda i: (i, 0)),
        ],
        out_shape=[
            jax.ShapeDtypeStruct((NPAD, 1), jnp.float32),
            jax.ShapeDtypeStruct((NPAD, D), jnp.float32),
        ],
    )(degp, xp, W1p)


def _tc_layer(agg, y, dinv, bp, Wnp, act):
    grid = NPAD // RB

    def body(agg_ref, y_ref, dinv_ref, b_ref, w_ref, yn_ref):
        a = agg_ref[0] + agg_ref[1] + y_ref[...]
        h = act(a * dinv_ref[...] + b_ref[...])
        yn_ref[...] = jnp.dot(h, w_ref[...]) * dinv_ref[...]

    return pl.pallas_call(
        body,
        grid=(grid,),
        in_specs=[
            pl.BlockSpec((2, RB, D), lambda i: (0, i, 0)),
            pl.BlockSpec((RB, D), lambda i: (i, 0)),
            pl.BlockSpec((RB, 1), lambda i: (i, 0)),
            pl.BlockSpec((1, D), lambda i: (0, 0)),
            pl.BlockSpec((D, D), lambda i: (0, 0)),
        ],
        out_specs=pl.BlockSpec((RB, D), lambda i: (i, 0)),
        out_shape=jax.ShapeDtypeStruct((NPAD, D), jnp.float32),
    )(agg, y, dinv, bp, Wnp)


def _tc_tail(agg, y, dinv, bp, Wcp, bc, n_out, d_out):
    dout = Wcp.shape[1]
    rbt = RB
    grid = NPAD // rbt

    def body(agg_ref, y_ref, dinv_ref, b_ref, wc_ref, bc_ref, h_ref, o_ref):
        a = agg_ref[0] + agg_ref[1] + y_ref[...]
        h = jnp.tanh(a * dinv_ref[...] + b_ref[...])
        h_ref[...] = h
        o_ref[...] = jnp.dot(h, wc_ref[...]) + bc_ref[...]

    return pl.pallas_call(
        body,
        grid=(grid,),
        in_specs=[
            pl.BlockSpec((2, rbt, D), lambda i: (0, i, 0)),
            pl.BlockSpec((rbt, D), lambda i: (i, 0)),
            pl.BlockSpec((rbt, 1), lambda i: (i, 0)),
            pl.BlockSpec((1, D), lambda i: (0, 0)),
            pl.BlockSpec((D, dout), lambda i: (0, 0)),
            pl.BlockSpec((1, dout), lambda i: (0, 0)),
        ],
        out_specs=[
            pl.BlockSpec((rbt, D), lambda i: (i, 0)),
            pl.BlockSpec((rbt, dout), lambda i: (i, 0)),
        ],
        out_shape=[
            jax.ShapeDtypeStruct((NPAD, D), jnp.float32),
            jax.ShapeDtypeStruct((NPAD, dout), jnp.float32),
        ],
    )(agg, y, dinv, bp, Wcp, bc.reshape(1, dout))


def _padw(W):
    return jnp.pad(W, ((0, D - W.shape[0]), (0, D - W.shape[1])))


def _padb(b):
    return jnp.pad(b, (0, D - b.shape[0])).reshape(1, D)


def kernel(x, edge_index, W1, b1, W2, b2, W3, b3, W4, b4, Wc, bc):
    N = x.shape[0]
    assert N <= NPAD
    ei = edge_index.astype(jnp.int32)
    src, dst = ei[0], ei[1]

    xp = jnp.pad(x, ((0, NPAD - N), (0, D - x.shape[1])))
    zeros_n = jnp.zeros((NPAD,), jnp.float32)
    zeros_nd = jnp.zeros((NPAD, D), jnp.float32)

    degp = _sc_degree(dst, zeros_n)
    dinv, y1 = _tc_head(degp, xp, _padw(W1))

    agg1 = _sc_edge_pass(src, dst, y1, zeros_nd, 4)
    y2 = _tc_layer(agg1, y1, dinv, _padb(b1), _padw(W2), jax.nn.relu)

    agg2 = _sc_edge_pass(src, dst, y2, zeros_nd, 4)
    y3 = _tc_layer(agg2, y2, dinv, _padb(b2), _padw(W3), jnp.tanh)

    agg3 = _sc_edge_pass(src, dst, y3, zeros_nd, 2)
    y4 = _tc_layer(agg3, y3, dinv, _padb(b3), _padw(W4), jax.nn.relu)

    agg4 = _sc_edge_pass(src, dst, y4, zeros_nd, 2)
    h4, out = _tc_tail(agg4, y4, dinv, _padb(b4),
                       jnp.pad(Wc, ((0, D - Wc.shape[0]), (0, 0))), bc,
                       N, W4.shape[1])

    return (out[:N], h4[:N, :W4.shape[1]])


# speedup vs baseline: 1.2200x; 1.2200x over previous
import functools

import jax
import jax.numpy as jnp
from jax import lax
from jax.experimental import pallas as pl
from jax.experimental.pallas import tpu as pltpu
from jax.experimental.pallas import tpu_sc as plsc

NC = 2
NS = 16
NPAD = 102400
RB = 6400
W_EDGE = 2000
D = 8


def _sc_mesh():
    return plsc.VectorSubcoreMesh(
        core_axis_name="c", subcore_axis_name="s", num_cores=NC,
        num_subcores=NS)


_SC_PARAMS = pltpu.CompilerParams(use_tc_tiling_on_sc=False)


def _sc_degree(dst, zeros_n):
    E = dst.shape[0]
    ept = E // (NC * NS)
    nwin = ept // W_EDGE
    assert nwin * W_EDGE == ept
    rpt = NPAD // NS

    assert nwin % 4 == 0 and nwin >= 8

    @functools.partial(
        pl.kernel,
        out_type=jax.ShapeDtypeStruct((NC, NPAD), jnp.float32),
        mesh=_sc_mesh(),
        scratch_types=[
            pltpu.VMEM_SHARED((NPAD,), jnp.float32),
            [pltpu.VMEM((W_EDGE,), jnp.int32) for _ in range(4)],
            pltpu.VMEM((W_EDGE,), jnp.float32),
            [pltpu.SemaphoreType.DMA for _ in range(4)],
            [pltpu.SemaphoreType.DMA for _ in range(4)],
        ],
        compiler_params=_SC_PARAMS,
    )
    def deg_kernel(dst_hbm, zeros_hbm, out_hbm, deg_s, dbufs, ones,
                   isems, ssems):
        c = lax.axis_index("c")
        s = lax.axis_index("s")
        sl = pl.ds(s * rpt, rpt)
        pltpu.sync_copy(zeros_hbm.at[sl], deg_s.at[sl])

        def fill(i, _):
            ones[pl.ds(i * 16, 16)] = jnp.full((16,), 1.0, jnp.float32)
            return 0
        lax.fori_loop(0, W_EDGE // 16, fill, 0)
        plsc.subcore_barrier()

        base0 = (c * NS + s) * ept

        def istart(w, p):
            b = base0 + w * W_EDGE
            pltpu.async_copy(dst_hbm.at[pl.ds(b, W_EDGE)], dbufs[p],
                             isems[p])

        def sdesc(p):
            return pltpu.make_async_copy(ones, deg_s.at[dbufs[p]], ssems[p])

        istart(0, 0)
        istart(1, 1)

        def grp(g, _):
            for ph in range(4):
                w = g * 4 + ph
                p = ph
                q = (ph + 2) % 4
                pltpu.make_async_copy(
                    dst_hbm.at[pl.ds(base0, W_EDGE)], dbufs[p],
                    isems[p]).wait()
                pltpu.async_copy(ones, deg_s.at[dbufs[p]], ssems[p],
                                 add=True)

                @pl.when(w >= 2)
                def _():
                    sdesc(q).wait()

                @pl.when(w + 2 < nwin)
                def _():
                    istart(w + 2, q)
            return 0
        lax.fori_loop(0, nwin // 4, grp, 0)
        sdesc((nwin - 2) % 4).wait()
        sdesc((nwin - 1) % 4).wait()
        plsc.subcore_barrier()
        pltpu.sync_copy(deg_s.at[sl], out_hbm.at[c, sl])

    return deg_kernel(dst, zeros_n)


def _sc_edge_pass(src, dst, y, zeros_nd, dpack):
    E = src.shape[0]
    ept = E // (NC * NS)
    nwin = ept // W_EDGE
    assert nwin * W_EDGE == ept
    WE = 1000
    nwin = ept // WE
    assert nwin * WE == ept
    rpt = NPAD // NS
    assert nwin % 4 == 0 and nwin >= 8

    @functools.partial(
        pl.kernel,
        out_type=jax.ShapeDtypeStruct((NC, NPAD, D), jnp.float32),
        mesh=_sc_mesh(),
        scratch_types=[
            pltpu.VMEM_SHARED((NPAD, D), jnp.float32),
            pltpu.VMEM_SHARED((NPAD, D), jnp.float32),
            [pltpu.VMEM((WE,), jnp.int32) for _ in range(4)],
            [pltpu.VMEM((WE,), jnp.int32) for _ in range(4)],
            [pltpu.VMEM((WE, D), jnp.float32) for _ in range(2)],
            [pltpu.SemaphoreType.DMA for _ in range(4)],
            [pltpu.SemaphoreType.DMA for _ in range(2)],
            [pltpu.SemaphoreType.DMA for _ in range(2)],
        ],
        compiler_params=_SC_PARAMS,
    )
    def edge_kernel(src_hbm, dst_hbm, y_hbm, zeros_hbm, out_hbm,
                    y_s, agg_s, sbufs, dbufs, rowbufs, isems, gsems, ssems):
        c = lax.axis_index("c")
        s = lax.axis_index("s")
        sl = pl.ds(s * rpt, rpt)
        pltpu.sync_copy(y_hbm.at[sl], y_s.at[sl])
        pltpu.sync_copy(zeros_hbm.at[sl], agg_s.at[sl])
        plsc.subcore_barrier()

        base0 = (c * NS + s) * ept

        def istart(w, p):
            b = base0 + w * WE
            pltpu.async_copy(src_hbm.at[pl.ds(b, WE)], sbufs[p], isems[p])
            pltpu.async_copy(dst_hbm.at[pl.ds(b, WE)], dbufs[p], isems[p])

        def iwait(p):
            pltpu.make_async_copy(
                src_hbm.at[pl.ds(base0, WE)], sbufs[p], isems[p]).wait()
            pltpu.make_async_copy(
                dst_hbm.at[pl.ds(base0, WE)], dbufs[p], isems[p]).wait()

        def sdesc(p2):
            return pltpu.make_async_copy(
                rowbufs[p2], agg_s.at[dbufs[p2]], ssems[p2])

        istart(0, 0)
        istart(1, 1)

        def grp(g, _):
            for ph in range(4):
                w = g * 4 + ph
                p2 = ph % 2
                q4 = (ph + 2) % 4
                iwait(ph)

                @pl.when(w >= 2)
                def _():
                    sdesc(p2).wait()

                @pl.when(w + 2 < nwin)
                def _():
                    istart(w + 2, q4)

                pltpu.async_copy(y_s.at[sbufs[ph]], rowbufs[p2],
                                 gsems[p2]).wait()
                pltpu.async_copy(rowbufs[p2], agg_s.at[dbufs[ph]],
                                 ssems[p2], add=True)
            return 0
        lax.fori_loop(0, nwin // 4, grp, 0)
        sdesc(0).wait()
        sdesc(1).wait()
        plsc.subcore_barrier()
        pltpu.sync_copy(agg_s.at[sl], out_hbm.at[c, sl])

    return edge_kernel(src, dst, y, zeros_nd)


def _tc_head(degp, xp, W1p):
    grid = NPAD // RB

    def body(degp_ref, x_ref, w_ref, dinv_ref, y_ref):
        ones2 = jnp.ones((2, 1), jnp.float32)
        deg = lax.dot_general(degp_ref[...], ones2,
                              (((0,), (0,)), ((), ()))) + 1.0
        dinv = lax.rsqrt(deg)
        dinv_ref[...] = dinv
        y_ref[...] = jnp.dot(x_ref[...], w_ref[...]) * dinv

    return pl.pallas_call(
        body,
        grid=(grid,),
        in_specs=[
            pl.BlockSpec((2, RB), lambda i: (0, i)),
            pl.BlockSpec((RB, D), lambda i: (i, 0)),
            pl.BlockSpec((D, D), lambda i: (0, 0)),
        ],
        out_specs=[
            pl.BlockSpec((RB, 1), lambda i: (i, 0)),
            pl.BlockSpec((RB, D), lambda i: (i, 0)),
        ],
        out_shape=[
            jax.ShapeDtypeStruct((NPAD, 1), jnp.float32),
            jax.ShapeDtypeStruct((NPAD, D), jnp.float32),
        ],
    )(degp, xp, W1p)


def _tc_layer(agg, y, dinv, bp, Wnp, act):
    grid = NPAD // RB

    def body(agg_ref, y_ref, dinv_ref, b_ref, w_ref, yn_ref):
        a = agg_ref[0] + agg_ref[1] + y_ref[...]
        h = act(a * dinv_ref[...] + b_ref[...])
        yn_ref[...] = jnp.dot(h, w_ref[...]) * dinv_ref[...]

    return pl.pallas_call(
        body,
        grid=(grid,),
        in_specs=[
            pl.BlockSpec((2, RB, D), lambda i: (0, i, 0)),
            pl.BlockSpec((RB, D), lambda i: (i, 0)),
            pl.BlockSpec((RB, 1), lambda i: (i, 0)),
            pl.BlockSpec((1, D), lambda i: (0, 0)),
            pl.BlockSpec((D, D), lambda i: (0, 0)),
        ],
        out_specs=pl.BlockSpec((RB, D), lambda i: (i, 0)),
        out_shape=jax.ShapeDtypeStruct((NPAD, D), jnp.float32),
    )(agg, y, dinv, bp, Wnp)


def _tc_tail(agg, y, dinv, bp, Wcp, bc, n_out, d_out):
    dout = Wcp.shape[1]
    rbt = RB
    grid = NPAD // rbt

    def body(agg_ref, y_ref, dinv_ref, b_ref, wc_ref, bc_ref, h_ref, o_ref):
        a = agg_ref[0] + agg_ref[1] + y_ref[...]
        h = jnp.tanh(a * dinv_ref[...] + b_ref[...])
        h_ref[...] = h
        o_ref[...] = jnp.dot(h, wc_ref[...]) + bc_ref[...]

    return pl.pallas_call(
        body,
        grid=(grid,),
        in_specs=[
            pl.BlockSpec((2, rbt, D), lambda i: (0, i, 0)),
            pl.BlockSpec((rbt, D), lambda i: (i, 0)),
            pl.BlockSpec((rbt, 1), lambda i: (i, 0)),
            pl.BlockSpec((1, D), lambda i: (0, 0)),
            pl.BlockSpec((D, dout), lambda i: (0, 0)),
            pl.BlockSpec((1, dout), lambda i: (0, 0)),
        ],
        out_specs=[
            pl.BlockSpec((rbt, D), lambda i: (i, 0)),
            pl.BlockSpec((rbt, dout), lambda i: (i, 0)),
        ],
        out_shape=[
            jax.ShapeDtypeStruct((NPAD, D), jnp.float32),
            jax.ShapeDtypeStruct((NPAD, dout), jnp.float32),
        ],
    )(agg, y, dinv, bp, Wcp, bc.reshape(1, dout))


def _padw(W):
    return jnp.pad(W, ((0, D - W.shape[0]), (0, D - W.shape[1])))


def _padb(b):
    return jnp.pad(b, (0, D - b.shape[0])).reshape(1, D)


def kernel(x, edge_index, W1, b1, W2, b2, W3, b3, W4, b4, Wc, bc):
    N = x.shape[0]
    assert N <= NPAD
    ei = edge_index.astype(jnp.int32)
    src, dst = ei[0], ei[1]

    xp = jnp.pad(x, ((0, NPAD - N), (0, D - x.shape[1])))
    zeros_n = jnp.zeros((NPAD,), jnp.float32)
    zeros_nd = jnp.zeros((NPAD, D), jnp.float32)

    degp = _sc_degree(dst, zeros_n)
    dinv, y1 = _tc_head(degp, xp, _padw(W1))

    agg1 = _sc_edge_pass(src, dst, y1, zeros_nd, 4)
    y2 = _tc_layer(agg1, y1, dinv, _padb(b1), _padw(W2), jax.nn.relu)

    agg2 = _sc_edge_pass(src, dst, y2, zeros_nd, 4)
    y3 = _tc_layer(agg2, y2, dinv, _padb(b2), _padw(W3), jnp.tanh)

    agg3 = _sc_edge_pass(src, dst, y3, zeros_nd, 2)
    y4 = _tc_layer(agg3, y3, dinv, _padb(b3), _padw(W4), jax.nn.relu)

    agg4 = _sc_edge_pass(src, dst, y4, zeros_nd, 2)
    h4, out = _tc_tail(agg4, y4, dinv, _padb(b4),
                       jnp.pad(Wc, ((0, D - Wc.shape[0]), (0, 0))), bc,
                       N, W4.shape[1])

    return (out[:N], h4[:N, :W4.shape[1]])
